# baseline (speedup 1.0000x reference)
import jax
import jax.numpy as jnp
from jax import lax
from jax.experimental import pallas as pl
from jax.experimental.pallas import tpu as pltpu
from jax.experimental.pallas import tpu_sc as plsc

N, L, D, NBINS = 4096, 64, 32, 256
NC, NS = 2, 16
NW = NC * NS
CPW = (L * D) // NW
LPW = CPW // D
G = CPW // 16
GPL = D // 16
CHUNK = 128
NCHUNK = N // CHUNK


def _start_chunk(src_hbm, wid, ch, buf, sem):
    return [
        pltpu.async_copy(
            src_hbm.at[pl.ds(ch * CHUNK, CHUNK), LPW * wid + j, :],
            buf.at[j],
            sem,
        )
        for j in range(LPW)
    ]


def _double_buffered(src_hbm, wid, bufs, sems, consume):
    copies = [None, None]
    copies[0] = _start_chunk(src_hbm, wid, 0, bufs[0], sems[0])
    for ch in range(NCHUNK):
        cur = ch % 2
        for c in copies[cur]:
            c.wait()
        if ch + 1 < NCHUNK:
            nxt = 1 - cur
            copies[nxt] = _start_chunk(src_hbm, wid, ch + 1, bufs[nxt],
                                       sems[nxt])
        consume(bufs[cur])


def _groups(buf, i):
    out = []
    for j in range(LPW):
        for h in range(GPL):
            out.append(buf[j, i, pl.ds(h * 16, 16)])
    return out


def _kernel_body(xr_hbm, xf_hbm, out_hbm, buf0, buf1, hist_r, hist_f,
                 loss_v, sem0, sem1):
    wid = lax.axis_index("c") * NS + lax.axis_index("s")
    iota = lax.iota(jnp.int32, 16)
    base = [(g * 16 + iota) * NBINS for g in range(G)]
    bufs, sems = [buf0, buf1], [sem0, sem1]

    carry0 = (
        tuple(jnp.full((16,), jnp.inf, jnp.float32) for _ in range(G)),
        tuple(jnp.full((16,), -jnp.inf, jnp.float32) for _ in range(G)),
    )
    state = [carry0]

    def mm_consume(buf):
        def mmbody(i, carry):
            mns_c, mxs_c = carry
            xs = _groups(buf, i)
            new_mn = tuple(jnp.minimum(mns_c[g], xs[g]) for g in range(G))
            new_mx = tuple(jnp.maximum(mxs_c[g], xs[g]) for g in range(G))
            return new_mn, new_mx

        state[0] = plsc.parallel_loop(0, CHUNK, unroll=4, carry=state[0])(
            mmbody
        )

    _double_buffered(xr_hbm, wid, bufs, sems, mm_consume)
    mns, mxs = state[0]

    lo, hi, scale = [], [], []
    for g in range(G):
        mn, mx = mns[g], mxs[g]
        same = jnp.abs(mx - mn) < 1e-10
        mx = jnp.where(same, mx + 1e-5, mx)
        mn = jnp.where(same, mn - 1e-5, mn)
        lo.append(mn)
        hi.append(mx)
        scale.append((1.0 / (mx - mn)) * jnp.float32(NBINS))

    zeros = jnp.zeros((16,), jnp.float32)

    @plsc.parallel_loop(0, CPW * NBINS // 16, unroll=4)
    def zbody(i):
        hist_r[pl.ds(i * 16, 16)] = zeros
        hist_f[pl.ds(i * 16, 16)] = zeros

    ones = jnp.ones((16,), jnp.float32)

    def hist_consume(hist):
        def consume(buf):
            @plsc.parallel_loop(0, CHUNK, unroll=2)
            def body(i):
                xs = _groups(buf, i)
                for g in range(G):
                    x = xs[g]
                    tb = (x - lo[g]) * scale[g]
                    tb = jnp.minimum(jnp.maximum(tb, 0.0),
                                     jnp.float32(NBINS - 1))
                    idx = tb.astype(jnp.int32)
                    within = (x >= lo[g]) & (x <= hi[g])
                    plsc.addupdate_scatter(
                        hist, [idx + base[g]], ones, mask=within
                    )

        return consume

    _double_buffered(xr_hbm, wid, bufs, sems, hist_consume(hist_r))
    _double_buffered(xf_hbm, wid, bufs, sems, hist_consume(hist_f))

    inv_n = jnp.float32(1.0 / N)
    for g in range(G):
        colbase = base[g]

        def fbody(b, carry):
            sa, sf = carry
            cr = plsc.load_gather(hist_r, [colbase + b])
            cf = plsc.load_gather(hist_f, [colbase + b])
            return sa + jnp.abs(cf - cr), sf + cf

        sa, sf = plsc.parallel_loop(0, NBINS, unroll=4, carry=(zeros, zeros))(
            fbody
        )
        loss_g = sa * inv_n + (jnp.float32(N) - sf) * inv_n
        bw = (hi[g] - lo[g]) * jnp.float32(1.0 / NBINS)
        c_first = lo[g] + bw * jnp.float32(0.5)
        c_last = lo[g] + bw * jnp.float32(NBINS - 0.5)
        deg = (jnp.abs(c_first) < 1e-16) & (jnp.abs(c_last) < 1e-16)
        loss_g = jnp.where(deg, jnp.float32(2.0), loss_g)
        loss_v[pl.ds(g * 16, 16)] = loss_g

    pltpu.sync_copy(loss_v, out_hbm.at[pl.ds(wid * CPW, CPW)])


@jax.jit
def _hist_loss(xr, xf):
    mesh = plsc.VectorSubcoreMesh(
        core_axis_name="c", subcore_axis_name="s", num_cores=NC, num_subcores=NS
    )
    return pl.kernel(
        _kernel_body,
        out_type=jax.ShapeDtypeStruct((L * D,), jnp.float32),
        mesh=mesh,
        compiler_params=pltpu.CompilerParams(needs_layout_passes=False),
        scratch_types=[
            pltpu.VMEM((LPW, CHUNK, D), jnp.float32),
            pltpu.VMEM((LPW, CHUNK, D), jnp.float32),
            pltpu.VMEM((CPW * NBINS,), jnp.float32),
            pltpu.VMEM((CPW * NBINS,), jnp.float32),
            pltpu.VMEM((CPW,), jnp.float32),
            pltpu.SemaphoreType.DMA,
            pltpu.SemaphoreType.DMA,
        ],
    )(xr, xf)


def kernel(x_real, x_fake, n_bins):
    del n_bins
    return _hist_loss(x_real, x_fake).reshape(L, D)


# speedup vs baseline: 1.6345x; 1.6345x over previous
import jax
import jax.numpy as jnp
from jax import lax
from jax.experimental import pallas as pl
from jax.experimental.pallas import tpu as pltpu
from jax.experimental.pallas import tpu_sc as plsc

N, L, D, NBINS = 4096, 64, 32, 256
NC, NS = 2, 16
NW = NC * NS
CPW = (L * D) // NW
LPW = CPW // D
G = CPW // 16
GPL = D // 16
CHUNK = 256
NCHUNK = N // CHUNK


def _start_chunk(src_hbm, wid, ch, buf, sem):
    return [
        pltpu.async_copy(
            src_hbm.at[pl.ds(ch * CHUNK, CHUNK), wid],
            buf,
            sem,
        )
    ]


def _double_buffered(src_hbm, wid, bufs, sems, consume):
    copies = [None, None]
    copies[0] = _start_chunk(src_hbm, wid, 0, bufs[0], sems[0])
    for ch in range(NCHUNK):
        cur = ch % 2
        for c in copies[cur]:
            c.wait()
        if ch + 1 < NCHUNK:
            nxt = 1 - cur
            copies[nxt] = _start_chunk(src_hbm, wid, ch + 1, bufs[nxt],
                                       sems[nxt])
        consume(bufs[cur])


def _groups(buf, i):
    return [buf[i, pl.ds(g * 16, 16)] for g in range(G)]


def _kernel_body(xr_hbm, xf_hbm, out_hbm, buf0, buf1, hist_r, hist_f,
                 loss_v, sem0, sem1):
    wid = lax.axis_index("c") * NS + lax.axis_index("s")
    iota = lax.iota(jnp.int32, 16)
    base = [(g * 16 + iota) * NBINS for g in range(G)]
    bufs, sems = [buf0, buf1], [sem0, sem1]

    carry0 = (
        tuple(jnp.full((16,), jnp.inf, jnp.float32) for _ in range(G)),
        tuple(jnp.full((16,), -jnp.inf, jnp.float32) for _ in range(G)),
    )
    state = [carry0]

    def mm_consume(buf):
        def mmbody(i, carry):
            mns_c, mxs_c = carry
            xs = _groups(buf, i)
            new_mn = tuple(jnp.minimum(mns_c[g], xs[g]) for g in range(G))
            new_mx = tuple(jnp.maximum(mxs_c[g], xs[g]) for g in range(G))
            return new_mn, new_mx

        state[0] = plsc.parallel_loop(0, CHUNK, unroll=4, carry=state[0])(
            mmbody
        )

    _double_buffered(xr_hbm, wid, bufs, sems, mm_consume)
    mns, mxs = state[0]

    lo, hi, scale = [], [], []
    for g in range(G):
        mn, mx = mns[g], mxs[g]
        same = jnp.abs(mx - mn) < 1e-10
        mx = jnp.where(same, mx + 1e-5, mx)
        mn = jnp.where(same, mn - 1e-5, mn)
        lo.append(mn)
        hi.append(mx)
        scale.append((1.0 / (mx - mn)) * jnp.float32(NBINS))

    zeros = jnp.zeros((16,), jnp.float32)

    @plsc.parallel_loop(0, CPW * NBINS // 16, unroll=4)
    def zbody(i):
        hist_r[pl.ds(i * 16, 16)] = zeros
        hist_f[pl.ds(i * 16, 16)] = zeros

    ones = jnp.ones((16,), jnp.float32)

    def hist_consume(hist):
        def consume(buf):
            @plsc.parallel_loop(0, CHUNK, unroll=2)
            def body(i):
                xs = _groups(buf, i)
                for g in range(G):
                    x = xs[g]
                    tb = (x - lo[g]) * scale[g]
                    tb = jnp.minimum(jnp.maximum(tb, 0.0),
                                     jnp.float32(NBINS - 1))
                    idx = tb.astype(jnp.int32)
                    within = (x >= lo[g]) & (x <= hi[g])
                    plsc.addupdate_scatter(
                        hist, [idx + base[g]], ones, mask=within
                    )

        return consume

    _double_buffered(xr_hbm, wid, bufs, sems, hist_consume(hist_r))
    _double_buffered(xf_hbm, wid, bufs, sems, hist_consume(hist_f))

    inv_n = jnp.float32(1.0 / N)
    for g in range(G):
        colbase = base[g]

        def fbody(b, carry):
            sa, sf = carry
            cr = plsc.load_gather(hist_r, [colbase + b])
            cf = plsc.load_gather(hist_f, [colbase + b])
            return sa + jnp.abs(cf - cr), sf + cf

        sa, sf = plsc.parallel_loop(0, NBINS, unroll=4, carry=(zeros, zeros))(
            fbody
        )
        loss_g = sa * inv_n + (jnp.float32(N) - sf) * inv_n
        bw = (hi[g] - lo[g]) * jnp.float32(1.0 / NBINS)
        c_first = lo[g] + bw * jnp.float32(0.5)
        c_last = lo[g] + bw * jnp.float32(NBINS - 0.5)
        deg = (jnp.abs(c_first) < 1e-16) & (jnp.abs(c_last) < 1e-16)
        loss_g = jnp.where(deg, jnp.float32(2.0), loss_g)
        loss_v[pl.ds(g * 16, 16)] = loss_g

    pltpu.sync_copy(loss_v, out_hbm.at[pl.ds(wid * CPW, CPW)])


@jax.jit
def _hist_loss(xr, xf):
    mesh = plsc.VectorSubcoreMesh(
        core_axis_name="c", subcore_axis_name="s", num_cores=NC, num_subcores=NS
    )
    return pl.kernel(
        _kernel_body,
        out_type=jax.ShapeDtypeStruct((L * D,), jnp.float32),
        mesh=mesh,
        compiler_params=pltpu.CompilerParams(needs_layout_passes=False),
        scratch_types=[
            pltpu.VMEM((CHUNK, CPW), jnp.float32),
            pltpu.VMEM((CHUNK, CPW), jnp.float32),
            pltpu.VMEM((CPW * NBINS,), jnp.float32),
            pltpu.VMEM((CPW * NBINS,), jnp.float32),
            pltpu.VMEM((CPW,), jnp.float32),
            pltpu.SemaphoreType.DMA,
            pltpu.SemaphoreType.DMA,
        ],
    )(xr, xf)


def kernel(x_real, x_fake, n_bins):
    del n_bins
    xr = x_real.reshape(N, NW, CPW)
    xf = x_fake.reshape(N, NW, CPW)
    return _hist_loss(xr, xf).reshape(L, D)
